# baseline (speedup 1.0000x reference)
import functools

import jax
import jax.numpy as jnp
from jax import lax
from jax.experimental import pallas as pl
from jax.experimental.pallas import tpu as pltpu
from jax.experimental.pallas import tpu_sc as plsc

_NC = 2
_NS = 16
_NW = _NC * _NS

_CHUNK = 80


def _gidx_body(n, off, et_ref, src_ref, o_ref):
    o_ref[...] = et_ref[...] * n + src_ref[...] + off


def _flat_gather_index(etypes, src, n, off):
    e = etypes.shape[0]
    cols = 512
    rows = e // cols
    et2 = etypes.reshape(rows, cols)
    src2 = src.reshape(rows, cols)
    out = pl.pallas_call(
        functools.partial(_gidx_body, n, off),
        out_shape=jax.ShapeDtypeStruct((rows, cols), jnp.int32),
    )(et2, src2)
    return out.reshape(e)


def _compose_body(wc_ref, wb_ref, o_ref):
    o_ref[...] = jnp.dot(wc_ref[...], wb_ref[...],
                         preferred_element_type=jnp.float32)


def _compose_w(wc, wb):
    b, d, _ = wb.shape
    r = wc.shape[0]
    wb_flat = wb.reshape(b, d * d)
    cols = 2048
    grid = (d * d // cols,)
    out = pl.pallas_call(
        _compose_body,
        grid=grid,
        in_specs=[
            pl.BlockSpec((r, b), lambda i: (0, 0)),
            pl.BlockSpec((b, cols), lambda i: (0, i)),
        ],
        out_specs=pl.BlockSpec((r, cols), lambda i: (0, i)),
        out_shape=jax.ShapeDtypeStruct((r, d * d), jnp.float32),
    )(wc, wb_flat)
    return out.reshape(r, d, d)


def _transform_body(h_ref, w_ref, o_ref):
    g = pl.program_id(0)

    @pl.when(g == 0)
    def _():
        o_ref[...] = jnp.zeros_like(o_ref)

    @pl.when(g > 0)
    def _():
        o_ref[...] = jnp.dot(h_ref[...], w_ref[0],
                             preferred_element_type=jnp.float32)


def _transform(h, w, nbk):
    n, d = h.shape
    r = w.shape[0]
    nb = n // nbk
    grid = (r * nb + 1,)
    out = pl.pallas_call(
        _transform_body,
        grid=grid,
        in_specs=[
            pl.BlockSpec((nbk, d),
                         lambda g: (jnp.maximum(g - 1, 0) // r, 0)),
            pl.BlockSpec((1, d, d),
                         lambda g: (jnp.maximum(g - 1, 0) % r, 0, 0)),
        ],
        out_specs=pl.BlockSpec(
            (nbk, d),
            lambda g: (jnp.where(g == 0, 0,
                                 1 + (jnp.maximum(g - 1, 0) % r) * nb
                                 + jnp.maximum(g - 1, 0) // r), 0)),
        out_shape=jax.ShapeDtypeStruct(((r * n) + nbk, d), jnp.float32),
    )(h, w)
    return out


def _combine_body(part_ref, h_ref, lw_ref, b_ref, o_ref):
    loop = jnp.dot(h_ref[...], lw_ref[...], preferred_element_type=jnp.float32)
    o_ref[...] = jnp.maximum(part_ref[0] + part_ref[1] + loop + b_ref[...], 0.0)


def _combine(part, h, loop_w, bias2d, nbk):
    n, d = h.shape
    nb = n // nbk
    return pl.pallas_call(
        _combine_body,
        grid=(nb,),
        in_specs=[
            pl.BlockSpec((2, nbk, d), lambda i: (0, i, 0)),
            pl.BlockSpec((nbk, d), lambda i: (i, 0)),
            pl.BlockSpec((d, d), lambda i: (0, 0)),
            pl.BlockSpec((1, d), lambda i: (0, 0)),
        ],
        out_specs=pl.BlockSpec((nbk, d), lambda i: (i, 0)),
        out_shape=jax.ShapeDtypeStruct((n, d), jnp.float32),
    )(part, h, loop_w, bias2d)


def _make_edge_agg(n, d, ep):
    epw = ep // _NW
    nchunk = epw // _CHUNK
    cpb = 16
    nblk = nchunk // cpb
    npt = (n // _NS) // 8 * 8
    rem = n - _NS * npt

    mesh = plsc.VectorSubcoreMesh(core_axis_name="c", subcore_axis_name="s")

    @functools.partial(
        pl.kernel,
        out_type=jax.ShapeDtypeStruct((_NC, n, d), jnp.float32),
        mesh=mesh,
        scratch_types=[
            pltpu.VMEM((cpb, _CHUNK), jnp.int32),
            pltpu.VMEM((cpb, _CHUNK), jnp.int32),
            pltpu.VMEM((_CHUNK, d), jnp.float32),
            pltpu.VMEM((_CHUNK, d), jnp.float32),
            pltpu.VMEM_SHARED((n, d), jnp.float32),
            pltpu.SemaphoreType.DMA,
            pltpu.SemaphoreType.DMA,
        ],
    )
    def edge_agg(hw_hbm, gidx_hbm, dst_hbm, zeros_hbm, out_hbm,
                 idx_v, dst_v, rows_a, rows_b, acc_v, sem_a, sem_b):
        c = lax.axis_index("c")
        s = lax.axis_index("s")
        wid = s * _NC + c

        pltpu.sync_copy(zeros_hbm.at[pl.ds(s * npt, npt)],
                        acc_v.at[pl.ds(s * npt, npt)])
        if rem:
            @pl.when(s == _NS - 1)
            def _():
                pltpu.sync_copy(zeros_hbm.at[pl.ds(_NS * npt, rem)],
                                acc_v.at[pl.ds(_NS * npt, rem)])
        plsc.subcore_barrier()

        def block(k, carry):
            pltpu.sync_copy(gidx_hbm.at[wid, pl.ds(k * cpb, cpb)], idx_v)
            pltpu.sync_copy(dst_hbm.at[wid, pl.ds(k * cpb, cpb)], dst_v)
            pltpu.async_copy(hw_hbm.at[idx_v.at[0]], rows_a, sem_a)

            def body(i, c2):
                j = 2 * i
                pltpu.async_copy(hw_hbm.at[idx_v.at[j + 1]], rows_b, sem_b)
                pltpu.make_async_copy(hw_hbm.at[idx_v.at[j]], rows_a,
                                      sem_a).wait()
                pltpu.sync_copy(rows_a, acc_v.at[dst_v.at[j]], add=True)

                @pl.when(j + 2 < cpb)
                def _():
                    pltpu.async_copy(hw_hbm.at[idx_v.at[j + 2]], rows_a, sem_a)

                pltpu.make_async_copy(hw_hbm.at[idx_v.at[j + 1]], rows_b,
                                      sem_b).wait()
                pltpu.sync_copy(rows_b, acc_v.at[dst_v.at[j + 1]], add=True)
                return c2

            lax.fori_loop(0, cpb // 2, body, 0)
            return carry

        lax.fori_loop(0, nblk, block, 0)

        plsc.subcore_barrier()
        pltpu.sync_copy(acc_v.at[pl.ds(s * npt, npt)],
                        out_hbm.at[c, pl.ds(s * npt, npt)])
        if rem:
            @pl.when(s == _NS - 1)
            def _():
                pltpu.sync_copy(acc_v.at[pl.ds(_NS * npt, rem)],
                                out_hbm.at[c, pl.ds(_NS * npt, rem)])

    return edge_agg


def _layer(h, gidx3, dst3, zeros_nd, wb, wc, loop_w, bias2d, edge_agg, nbk):
    w = _compose_w(wc, wb)
    hw = _transform(h, w, nbk)
    part = edge_agg(hw, gidx3, dst3, zeros_nd)
    return _combine(part, h, loop_w, bias2d, nbk)


def kernel(features, edge_index, etypes, Wb1, wc1, loopW1, b1,
           Wb2, wc2, loopW2, b2):
    n, d = features.shape
    e = etypes.shape[0]
    src = edge_index[0]
    dst = edge_index[1]

    nbk = 5000
    cpb = 16
    epw = -(-(e // _NW) // (_CHUNK * cpb)) * (_CHUNK * cpb)
    ep = epw * _NW
    npad = ep - e

    gidx = _flat_gather_index(etypes, src, n, nbk)
    gidx3 = jnp.concatenate(
        [gidx, jnp.zeros((npad,), jnp.int32)]).reshape(_NW, -1, _CHUNK)
    dst3 = jnp.concatenate(
        [dst, jnp.arange(npad, dtype=jnp.int32) % n]).reshape(_NW, -1, _CHUNK)
    zeros_nd = jnp.zeros((n, d), jnp.float32)

    edge_agg = _make_edge_agg(n, d, ep)

    h1 = _layer(features, gidx3, dst3, zeros_nd, Wb1, wc1, loopW1,
                b1.reshape(1, d), edge_agg, nbk)
    h2 = _layer(h1, gidx3, dst3, zeros_nd, Wb2, wc2, loopW2,
                b2.reshape(1, d), edge_agg, nbk)
    return h2


# speedup vs baseline: 2.9715x; 2.9715x over previous
import functools

import jax
import jax.numpy as jnp
from jax import lax
from jax.experimental import pallas as pl
from jax.experimental.pallas import tpu as pltpu
from jax.experimental.pallas import tpu_sc as plsc

_NC = 2
_NS = 16
_NW = _NC * _NS

_CHUNK = 128


def _gidx_body(n, off, et_ref, src_ref, o_ref):
    o_ref[...] = et_ref[...] * n + src_ref[...] + off


def _flat_gather_index(etypes, src, n, off):
    e = etypes.shape[0]
    cols = 512
    rows = e // cols
    et2 = etypes.reshape(rows, cols)
    src2 = src.reshape(rows, cols)
    out = pl.pallas_call(
        functools.partial(_gidx_body, n, off),
        out_shape=jax.ShapeDtypeStruct((rows, cols), jnp.int32),
    )(et2, src2)
    return out.reshape(e)


def _compose_body(wc_ref, wb_ref, o_ref):
    o_ref[...] = jnp.dot(wc_ref[...], wb_ref[...],
                         preferred_element_type=jnp.float32)


def _compose_w(wc, wb):
    b, d, _ = wb.shape
    r = wc.shape[0]
    wb_flat = wb.reshape(b, d * d)
    cols = 2048
    grid = (d * d // cols,)
    out = pl.pallas_call(
        _compose_body,
        grid=grid,
        in_specs=[
            pl.BlockSpec((r, b), lambda i: (0, 0)),
            pl.BlockSpec((b, cols), lambda i: (0, i)),
        ],
        out_specs=pl.BlockSpec((r, cols), lambda i: (0, i)),
        out_shape=jax.ShapeDtypeStruct((r, d * d), jnp.float32),
    )(wc, wb_flat)
    return out.reshape(r, d, d)


def _transform_body(h_ref, w_ref, o_ref):
    g = pl.program_id(0)

    @pl.when(g == 0)
    def _():
        o_ref[...] = jnp.zeros_like(o_ref)

    @pl.when(g > 0)
    def _():
        o_ref[...] = jnp.dot(h_ref[...], w_ref[0],
                             preferred_element_type=jnp.float32)


def _transform(h, w, nbk):
    n, d = h.shape
    r = w.shape[0]
    nb = n // nbk
    grid = (r * nb + 1,)
    out = pl.pallas_call(
        _transform_body,
        grid=grid,
        in_specs=[
            pl.BlockSpec((nbk, d),
                         lambda g: (jnp.maximum(g - 1, 0) // r, 0)),
            pl.BlockSpec((1, d, d),
                         lambda g: (jnp.maximum(g - 1, 0) % r, 0, 0)),
        ],
        out_specs=pl.BlockSpec(
            (nbk, d),
            lambda g: (jnp.where(g == 0, 0,
                                 1 + (jnp.maximum(g - 1, 0) % r) * nb
                                 + jnp.maximum(g - 1, 0) // r), 0)),
        out_shape=jax.ShapeDtypeStruct(((r * n) + nbk, d), jnp.float32),
    )(h, w)
    return out


def _combine_body(part_ref, h_ref, lw_ref, b_ref, o_ref):
    loop = jnp.dot(h_ref[...], lw_ref[...], preferred_element_type=jnp.float32)
    o_ref[...] = jnp.maximum(part_ref[0] + part_ref[1] + loop + b_ref[...], 0.0)


def _combine(part, h, loop_w, bias2d, nbk):
    n, d = h.shape
    nb = n // nbk
    return pl.pallas_call(
        _combine_body,
        grid=(nb,),
        in_specs=[
            pl.BlockSpec((2, nbk, d), lambda i: (0, i, 0)),
            pl.BlockSpec((nbk, d), lambda i: (i, 0)),
            pl.BlockSpec((d, d), lambda i: (0, 0)),
            pl.BlockSpec((1, d), lambda i: (0, 0)),
        ],
        out_specs=pl.BlockSpec((nbk, d), lambda i: (i, 0)),
        out_shape=jax.ShapeDtypeStruct((n, d), jnp.float32),
    )(part, h, loop_w, bias2d)


def _make_edge_agg(n, d, ep):
    epw = ep // _NW
    nchunk = epw // _CHUNK
    cpb = 16
    nblk = nchunk // cpb
    npt = (n // _NS) // 8 * 8
    rem = n - _NS * npt

    mesh = plsc.VectorSubcoreMesh(core_axis_name="c", subcore_axis_name="s")

    @functools.partial(
        pl.kernel,
        out_type=jax.ShapeDtypeStruct((_NC, n, d), jnp.float32),
        mesh=mesh,
        scratch_types=[
            pltpu.VMEM((cpb, _CHUNK), jnp.int32),
            pltpu.VMEM((cpb, _CHUNK), jnp.int32),
            pltpu.VMEM((_CHUNK, d), jnp.float32),
            pltpu.VMEM((_CHUNK, d), jnp.float32),
            pltpu.VMEM_SHARED((n, d), jnp.float32),
            pltpu.SemaphoreType.DMA,
            pltpu.SemaphoreType.DMA,
        ],
    )
    def edge_agg(hw_hbm, gidx_hbm, dst_hbm, zeros_hbm, out_hbm,
                 idx_v, dst_v, rows_a, rows_b, acc_v, sem_a, sem_b):
        c = lax.axis_index("c")
        s = lax.axis_index("s")
        wid = s * _NC + c

        pltpu.sync_copy(zeros_hbm.at[pl.ds(s * npt, npt)],
                        acc_v.at[pl.ds(s * npt, npt)])
        if rem:
            @pl.when(s == _NS - 1)
            def _():
                pltpu.sync_copy(zeros_hbm.at[pl.ds(_NS * npt, rem)],
                                acc_v.at[pl.ds(_NS * npt, rem)])
        plsc.subcore_barrier()

        def block(k, carry):
            pltpu.sync_copy(gidx_hbm.at[wid, pl.ds(k * cpb, cpb)], idx_v)
            pltpu.sync_copy(dst_hbm.at[wid, pl.ds(k * cpb, cpb)], dst_v)
            pltpu.async_copy(hw_hbm.at[idx_v.at[0]], rows_a, sem_a)

            def body(i, c2):
                j = 2 * i
                pltpu.async_copy(hw_hbm.at[idx_v.at[j + 1]], rows_b, sem_b)
                pltpu.make_async_copy(hw_hbm.at[idx_v.at[j]], rows_a,
                                      sem_a).wait()
                pltpu.sync_copy(rows_a, acc_v.at[dst_v.at[j]], add=True)

                @pl.when(j + 2 < cpb)
                def _():
                    pltpu.async_copy(hw_hbm.at[idx_v.at[j + 2]], rows_a, sem_a)

                pltpu.make_async_copy(hw_hbm.at[idx_v.at[j + 1]], rows_b,
                                      sem_b).wait()
                pltpu.sync_copy(rows_b, acc_v.at[dst_v.at[j + 1]], add=True)
                return c2

            lax.fori_loop(0, cpb // 2, body, 0)
            return carry

        lax.fori_loop(0, nblk, block, 0)

        plsc.subcore_barrier()
        pltpu.sync_copy(acc_v.at[pl.ds(s * npt, npt)],
                        out_hbm.at[c, pl.ds(s * npt, npt)])
        if rem:
            @pl.when(s == _NS - 1)
            def _():
                pltpu.sync_copy(acc_v.at[pl.ds(_NS * npt, rem)],
                                out_hbm.at[c, pl.ds(_NS * npt, rem)])

    return edge_agg


def _layer(h, gidx3, dst3, zeros_nd, wb, wc, loop_w, bias2d, edge_agg, nbk):
    w = _compose_w(wc, wb)
    hw = _transform(h, w, nbk)
    part = edge_agg(hw, gidx3, dst3, zeros_nd)
    return _combine(part, h, loop_w, bias2d, nbk)


def kernel(features, edge_index, etypes, Wb1, wc1, loopW1, b1,
           Wb2, wc2, loopW2, b2):
    n, d = features.shape
    e = etypes.shape[0]
    src = edge_index[0]
    dst = edge_index[1]

    nbk = 5000
    cpb = 16
    epw = -(-(e // _NW) // (_CHUNK * cpb)) * (_CHUNK * cpb)
    ep = epw * _NW
    npad = ep - e

    gidx = _flat_gather_index(etypes, src, n, nbk)
    ppw = npad // _NW
    padg = (jnp.arange(npad, dtype=jnp.int32) % nbk).reshape(_NW, ppw)
    padd = (jnp.arange(npad, dtype=jnp.int32) % n).reshape(_NW, ppw)
    gidx3 = jnp.concatenate(
        [gidx.reshape(_NW, -1), padg], axis=1).reshape(_NW, -1, _CHUNK)
    dst3 = jnp.concatenate(
        [dst.reshape(_NW, -1), padd], axis=1).reshape(_NW, -1, _CHUNK)
    zeros_nd = jnp.zeros((n, d), jnp.float32)

    edge_agg = _make_edge_agg(n, d, ep)

    h1 = _layer(features, gidx3, dst3, zeros_nd, Wb1, wc1, loopW1,
                b1.reshape(1, d), edge_agg, nbk)
    h2 = _layer(h1, gidx3, dst3, zeros_nd, Wb2, wc2, loopW2,
                b2.reshape(1, d), edge_agg, nbk)
    return h2


# speedup vs baseline: 2.9750x; 1.0012x over previous
import functools

import jax
import jax.numpy as jnp
from jax import lax
from jax.experimental import pallas as pl
from jax.experimental.pallas import tpu as pltpu
from jax.experimental.pallas import tpu_sc as plsc

_NC = 2
_NS = 16
_NW = _NC * _NS

_CHUNK = 128


def _gidx_body(n, off, et_ref, src_ref, o_ref):
    o_ref[...] = et_ref[...] * n + src_ref[...] + off


def _flat_gather_index(etypes, src, n, off):
    e = etypes.shape[0]
    cols = 512
    rows = e // cols
    et2 = etypes.reshape(rows, cols)
    src2 = src.reshape(rows, cols)
    out = pl.pallas_call(
        functools.partial(_gidx_body, n, off),
        out_shape=jax.ShapeDtypeStruct((rows, cols), jnp.int32),
    )(et2, src2)
    return out.reshape(e)


def _compose_body(wc_ref, wb_ref, o_ref):
    o_ref[...] = jnp.dot(wc_ref[...], wb_ref[...],
                         preferred_element_type=jnp.float32)


def _compose_w(wc, wb):
    b, d, _ = wb.shape
    r = wc.shape[0]
    wb_flat = wb.reshape(b, d * d)
    cols = 2048
    grid = (d * d // cols,)
    out = pl.pallas_call(
        _compose_body,
        grid=grid,
        in_specs=[
            pl.BlockSpec((r, b), lambda i: (0, 0)),
            pl.BlockSpec((b, cols), lambda i: (0, i)),
        ],
        out_specs=pl.BlockSpec((r, cols), lambda i: (0, i)),
        out_shape=jax.ShapeDtypeStruct((r, d * d), jnp.float32),
    )(wc, wb_flat)
    return out.reshape(r, d, d)


def _transform_body(h_ref, w_ref, o_ref):
    g = pl.program_id(0)

    @pl.when(g == 0)
    def _():
        o_ref[...] = jnp.zeros_like(o_ref)

    @pl.when(g > 0)
    def _():
        o_ref[...] = jnp.dot(h_ref[...], w_ref[0],
                             preferred_element_type=jnp.float32)


def _transform(h, w, nbk):
    n, d = h.shape
    r = w.shape[0]
    nb = n // nbk
    grid = (r * nb + 1,)
    out = pl.pallas_call(
        _transform_body,
        grid=grid,
        in_specs=[
            pl.BlockSpec((nbk, d),
                         lambda g: (jnp.maximum(g - 1, 0) // r, 0)),
            pl.BlockSpec((1, d, d),
                         lambda g: (jnp.maximum(g - 1, 0) % r, 0, 0)),
        ],
        out_specs=pl.BlockSpec(
            (nbk, d),
            lambda g: (jnp.where(g == 0, 0,
                                 1 + (jnp.maximum(g - 1, 0) % r) * nb
                                 + jnp.maximum(g - 1, 0) // r), 0)),
        out_shape=jax.ShapeDtypeStruct(((r * n) + nbk, d), jnp.float32),
    )(h, w)
    return out


def _combine_body(part_ref, h_ref, lw_ref, b_ref, o_ref):
    loop = jnp.dot(h_ref[...], lw_ref[...], preferred_element_type=jnp.float32)
    o_ref[...] = jnp.maximum(part_ref[0] + part_ref[1] + loop + b_ref[...], 0.0)


def _combine(part, h, loop_w, bias2d, nbk):
    n, d = h.shape
    nb = n // nbk
    return pl.pallas_call(
        _combine_body,
        grid=(nb,),
        in_specs=[
            pl.BlockSpec((2, nbk, d), lambda i: (0, i, 0)),
            pl.BlockSpec((nbk, d), lambda i: (i, 0)),
            pl.BlockSpec((d, d), lambda i: (0, 0)),
            pl.BlockSpec((1, d), lambda i: (0, 0)),
        ],
        out_specs=pl.BlockSpec((nbk, d), lambda i: (i, 0)),
        out_shape=jax.ShapeDtypeStruct((n, d), jnp.float32),
    )(part, h, loop_w, bias2d)


def _make_edge_agg(n, d, ep):
    epw = ep // _NW
    nchunk = epw // _CHUNK
    cpb = 16
    nblk = nchunk // cpb
    npt = (n // _NS) // 8 * 8
    rem = n - _NS * npt

    mesh = plsc.VectorSubcoreMesh(core_axis_name="c", subcore_axis_name="s")

    @functools.partial(
        pl.kernel,
        out_type=jax.ShapeDtypeStruct((_NC, n, d), jnp.float32),
        mesh=mesh,
        scratch_types=[
            pltpu.VMEM((cpb, _CHUNK), jnp.int32),
            pltpu.VMEM((cpb, _CHUNK), jnp.int32),
            pltpu.VMEM((_CHUNK, d), jnp.float32),
            pltpu.VMEM((_CHUNK, d), jnp.float32),
            pltpu.VMEM_SHARED((n, d), jnp.float32),
            pltpu.SemaphoreType.DMA,
            pltpu.SemaphoreType.DMA,
        ],
    )
    def edge_agg(hw_hbm, gidx_hbm, dst_hbm, out_hbm,
                 idx_v, dst_v, rows_a, rows_b, acc_v, sem_a, sem_b):
        c = lax.axis_index("c")
        s = lax.axis_index("s")
        wid = s * _NC + c

        pltpu.sync_copy(hw_hbm.at[pl.ds(0, npt)],
                        acc_v.at[pl.ds(s * npt, npt)])
        if rem:
            @pl.when(s == _NS - 1)
            def _():
                pltpu.sync_copy(hw_hbm.at[pl.ds(0, rem)],
                                acc_v.at[pl.ds(_NS * npt, rem)])
        plsc.subcore_barrier()

        def block(k, carry):
            pltpu.sync_copy(gidx_hbm.at[wid, pl.ds(k * cpb, cpb)], idx_v)
            pltpu.sync_copy(dst_hbm.at[wid, pl.ds(k * cpb, cpb)], dst_v)
            pltpu.async_copy(hw_hbm.at[idx_v.at[0]], rows_a, sem_a)

            def body(i, c2):
                j = 2 * i
                pltpu.async_copy(hw_hbm.at[idx_v.at[j + 1]], rows_b, sem_b)
                pltpu.make_async_copy(hw_hbm.at[idx_v.at[j]], rows_a,
                                      sem_a).wait()
                pltpu.sync_copy(rows_a, acc_v.at[dst_v.at[j]], add=True)

                @pl.when(j + 2 < cpb)
                def _():
                    pltpu.async_copy(hw_hbm.at[idx_v.at[j + 2]], rows_a, sem_a)

                pltpu.make_async_copy(hw_hbm.at[idx_v.at[j + 1]], rows_b,
                                      sem_b).wait()
                pltpu.sync_copy(rows_b, acc_v.at[dst_v.at[j + 1]], add=True)
                return c2

            lax.fori_loop(0, cpb // 2, body, 0)
            return carry

        lax.fori_loop(0, nblk, block, 0)

        plsc.subcore_barrier()
        pltpu.sync_copy(acc_v.at[pl.ds(s * npt, npt)],
                        out_hbm.at[c, pl.ds(s * npt, npt)])
        if rem:
            @pl.when(s == _NS - 1)
            def _():
                pltpu.sync_copy(acc_v.at[pl.ds(_NS * npt, rem)],
                                out_hbm.at[c, pl.ds(_NS * npt, rem)])

    return edge_agg


def _layer(h, gidx3, dst3, wb, wc, loop_w, bias2d, edge_agg, nbk):
    w = _compose_w(wc, wb)
    hw = _transform(h, w, nbk)
    part = edge_agg(hw, gidx3, dst3)
    return _combine(part, h, loop_w, bias2d, nbk)


def kernel(features, edge_index, etypes, Wb1, wc1, loopW1, b1,
           Wb2, wc2, loopW2, b2):
    n, d = features.shape
    e = etypes.shape[0]
    src = edge_index[0]
    dst = edge_index[1]

    nbk = 5000
    cpb = 16
    epw = -(-(e // _NW) // (_CHUNK * cpb)) * (_CHUNK * cpb)
    ep = epw * _NW
    npad = ep - e

    gidx = _flat_gather_index(etypes, src, n, nbk)
    padg = jnp.arange(npad, dtype=jnp.int32) % nbk
    padd = jnp.arange(npad, dtype=jnp.int32) % n
    gidx3 = jnp.concatenate([gidx, padg]).reshape(_NW, -1, _CHUNK)
    dst3 = jnp.concatenate([dst, padd]).reshape(_NW, -1, _CHUNK)

    edge_agg = _make_edge_agg(n, d, ep)

    h1 = _layer(features, gidx3, dst3, Wb1, wc1, loopW1,
                b1.reshape(1, d), edge_agg, nbk)
    h2 = _layer(h1, gidx3, dst3, Wb2, wc2, loopW2,
                b2.reshape(1, d), edge_agg, nbk)
    return h2


# speedup vs baseline: 3.1867x; 1.0711x over previous
import functools

import jax
import jax.numpy as jnp
from jax import lax
from jax.experimental import pallas as pl
from jax.experimental.pallas import tpu as pltpu
from jax.experimental.pallas import tpu_sc as plsc

_NC = 2
_NS = 16
_NW = _NC * _NS

_CHUNK = 64
_NBUF = 4


def _gidx_body(n, off, et_ref, src_ref, o_ref):
    o_ref[...] = et_ref[...] * n + src_ref[...] + off


def _flat_gather_index(etypes, src, n, off):
    e = etypes.shape[0]
    cols = 512
    rows = e // cols
    et2 = etypes.reshape(rows, cols)
    src2 = src.reshape(rows, cols)
    out = pl.pallas_call(
        functools.partial(_gidx_body, n, off),
        out_shape=jax.ShapeDtypeStruct((rows, cols), jnp.int32),
    )(et2, src2)
    return out.reshape(e)


def _compose_body(wc_ref, wb_ref, o_ref):
    o_ref[...] = jnp.dot(wc_ref[...], wb_ref[...],
                         preferred_element_type=jnp.float32)


def _compose_w(wc, wb):
    b, d, _ = wb.shape
    r = wc.shape[0]
    wb_flat = wb.reshape(b, d * d)
    cols = 2048
    grid = (d * d // cols,)
    out = pl.pallas_call(
        _compose_body,
        grid=grid,
        in_specs=[
            pl.BlockSpec((r, b), lambda i: (0, 0)),
            pl.BlockSpec((b, cols), lambda i: (0, i)),
        ],
        out_specs=pl.BlockSpec((r, cols), lambda i: (0, i)),
        out_shape=jax.ShapeDtypeStruct((r, d * d), jnp.float32),
    )(wc, wb_flat)
    return out.reshape(r, d, d)


def _transform_body(h_ref, w_ref, o_ref):
    g = pl.program_id(0)

    @pl.when(g == 0)
    def _():
        o_ref[...] = jnp.zeros_like(o_ref)

    @pl.when(g > 0)
    def _():
        o_ref[...] = jnp.dot(h_ref[...], w_ref[0],
                             preferred_element_type=jnp.float32)


def _transform(h, w, nbk):
    n, d = h.shape
    r = w.shape[0]
    nb = n // nbk
    grid = (r * nb + 1,)
    out = pl.pallas_call(
        _transform_body,
        grid=grid,
        in_specs=[
            pl.BlockSpec((nbk, d),
                         lambda g: (jnp.maximum(g - 1, 0) // r, 0)),
            pl.BlockSpec((1, d, d),
                         lambda g: (jnp.maximum(g - 1, 0) % r, 0, 0)),
        ],
        out_specs=pl.BlockSpec(
            (nbk, d),
            lambda g: (jnp.where(g == 0, 0,
                                 1 + (jnp.maximum(g - 1, 0) % r) * nb
                                 + jnp.maximum(g - 1, 0) // r), 0)),
        out_shape=jax.ShapeDtypeStruct(((r * n) + nbk, d), jnp.float32),
    )(h, w)
    return out


def _combine_body(part_ref, h_ref, lw_ref, b_ref, o_ref):
    loop = jnp.dot(h_ref[...], lw_ref[...], preferred_element_type=jnp.float32)
    o_ref[...] = jnp.maximum(part_ref[0] + part_ref[1] + loop + b_ref[...], 0.0)


def _combine(part, h, loop_w, bias2d, nbk):
    n, d = h.shape
    nb = n // nbk
    return pl.pallas_call(
        _combine_body,
        grid=(nb,),
        in_specs=[
            pl.BlockSpec((2, nbk, d), lambda i: (0, i, 0)),
            pl.BlockSpec((nbk, d), lambda i: (i, 0)),
            pl.BlockSpec((d, d), lambda i: (0, 0)),
            pl.BlockSpec((1, d), lambda i: (0, 0)),
        ],
        out_specs=pl.BlockSpec((nbk, d), lambda i: (i, 0)),
        out_shape=jax.ShapeDtypeStruct((n, d), jnp.float32),
    )(part, h, loop_w, bias2d)


def _make_edge_agg(n, d, ep):
    epw = ep // _NW
    nchunk = epw // _CHUNK
    cpb = 32
    nblk = nchunk // cpb
    npt = (n // _NS) // 8 * 8
    rem = n - _NS * npt

    mesh = plsc.VectorSubcoreMesh(core_axis_name="c", subcore_axis_name="s")

    @functools.partial(
        pl.kernel,
        out_type=jax.ShapeDtypeStruct((_NC, n, d), jnp.float32),
        mesh=mesh,
        scratch_types=[
            pltpu.VMEM((cpb, _CHUNK), jnp.int32),
            pltpu.VMEM((cpb, _CHUNK), jnp.int32),
            [pltpu.VMEM((_CHUNK, d), jnp.float32) for _ in range(_NBUF)],
            pltpu.VMEM_SHARED((n, d), jnp.float32),
            [pltpu.SemaphoreType.DMA for _ in range(_NBUF)],
        ],
    )
    def edge_agg(hw_hbm, gidx_hbm, dst_hbm, out_hbm,
                 idx_v, dst_v, rows, acc_v, sems):
        c = lax.axis_index("c")
        s = lax.axis_index("s")
        wid = s * _NC + c

        pltpu.sync_copy(hw_hbm.at[pl.ds(0, npt)],
                        acc_v.at[pl.ds(s * npt, npt)])
        if rem:
            @pl.when(s == _NS - 1)
            def _():
                pltpu.sync_copy(hw_hbm.at[pl.ds(0, rem)],
                                acc_v.at[pl.ds(_NS * npt, rem)])
        plsc.subcore_barrier()

        def block(k, carry):
            pltpu.sync_copy(gidx_hbm.at[wid, pl.ds(k * cpb, cpb)], idx_v)
            pltpu.sync_copy(dst_hbm.at[wid, pl.ds(k * cpb, cpb)], dst_v)
            for u in range(_NBUF - 1):
                pltpu.async_copy(hw_hbm.at[idx_v.at[u]], rows[u], sems[u])

            def group(g2, c2):
                base = g2 * _NBUF
                for u in range(_NBUF):
                    j = base + u
                    nxt = (u + _NBUF - 1) % _NBUF

                    @pl.when(j + _NBUF - 1 < cpb)
                    def _():
                        pltpu.async_copy(hw_hbm.at[idx_v.at[j + _NBUF - 1]],
                                         rows[nxt], sems[nxt])

                    pltpu.make_async_copy(hw_hbm.at[idx_v.at[j]], rows[u],
                                          sems[u]).wait()
                    pltpu.sync_copy(rows[u], acc_v.at[dst_v.at[j]], add=True)
                return c2

            lax.fori_loop(0, cpb // _NBUF, group, 0)
            return carry

        lax.fori_loop(0, nblk, block, 0)

        plsc.subcore_barrier()
        pltpu.sync_copy(acc_v.at[pl.ds(s * npt, npt)],
                        out_hbm.at[c, pl.ds(s * npt, npt)])
        if rem:
            @pl.when(s == _NS - 1)
            def _():
                pltpu.sync_copy(acc_v.at[pl.ds(_NS * npt, rem)],
                                out_hbm.at[c, pl.ds(_NS * npt, rem)])

    return edge_agg


def _layer(h, gidx3, dst3, wb, wc, loop_w, bias2d, edge_agg, nbk):
    w = _compose_w(wc, wb)
    hw = _transform(h, w, nbk)
    part = edge_agg(hw, gidx3, dst3)
    return _combine(part, h, loop_w, bias2d, nbk)


def kernel(features, edge_index, etypes, Wb1, wc1, loopW1, b1,
           Wb2, wc2, loopW2, b2):
    n, d = features.shape
    e = etypes.shape[0]
    src = edge_index[0]
    dst = edge_index[1]

    nbk = 5000
    cpb = 32
    epw = -(-(e // _NW) // (_CHUNK * cpb)) * (_CHUNK * cpb)
    ep = epw * _NW
    npad = ep - e

    gidx = _flat_gather_index(etypes, src, n, nbk)
    padg = jnp.arange(npad, dtype=jnp.int32) % nbk
    padd = jnp.arange(npad, dtype=jnp.int32) % n
    gidx3 = jnp.concatenate([gidx, padg]).reshape(_NW, -1, _CHUNK)
    dst3 = jnp.concatenate([dst, padd]).reshape(_NW, -1, _CHUNK)

    edge_agg = _make_edge_agg(n, d, ep)

    h1 = _layer(features, gidx3, dst3, Wb1, wc1, loopW1,
                b1.reshape(1, d), edge_agg, nbk)
    h2 = _layer(h1, gidx3, dst3, Wb2, wc2, loopW2,
                b2.reshape(1, d), edge_agg, nbk)
    return h2
